# Initial kernel scaffold; baseline (speedup 1.0000x reference)
import jax
import jax.numpy as jnp
from jax.experimental import pallas as pl


def kernel(left_features, right_features, edge_index, W):
    raise NotImplementedError("write your pallas kernel here")


# speedup vs baseline: 5.5680x; 5.5680x over previous
import functools

import jax
import jax.numpy as jnp
from jax import lax
from jax.experimental import pallas as pl
from jax.experimental.pallas import tpu as pltpu
from jax.experimental.pallas import tpu_sc as plsc

N_NODES = 10000
N_EDGES = 320000
D = 128
NUM_HEADS = 2
THRESH = 0.1
EPS = 1e-8

NC = 2
NS = 16
NW = NC * NS
CPW = N_EDGES // NW
C = 80
NCH = CPW // C
NG = C // 16


def _prep_body(l_ref, r_ref, w_ref, rln_ref, rrn_ref, wsq_ref):
    w = w_ref[...]
    wsq_ref[...] = w * w
    outs = []
    for ref in (l_ref, r_ref):
        x = ref[...]
        cols = []
        for h in range(NUM_HEADS):
            xh = x * w[h:h + 1, :]
            s = jnp.sum(xh * xh, axis=1, keepdims=True)
            cols.append(lax.rsqrt(jnp.maximum(s, EPS * EPS)))
        outs.append(jnp.concatenate(cols, axis=1))
    rln_ref[...] = outs[0]
    rrn_ref[...] = outs[1]


def _prep(left, right, w):
    return pl.pallas_call(
        _prep_body,
        out_shape=(
            jax.ShapeDtypeStruct((N_NODES, NUM_HEADS), jnp.float32),
            jax.ShapeDtypeStruct((N_NODES, NUM_HEADS), jnp.float32),
            jax.ShapeDtypeStruct((NUM_HEADS, D), jnp.float32),
        ),
    )(left, right, w)


def _sc_body(left_hbm, right_hbm, src_hbm, dst_hbm, rln_hbm, rrn_hbm, wsq_hbm,
             out_hbm, src_v, dst_v, out_v, rln_v, rrn_v, wsq_v, lrows, rrows,
             lsem, rsem):
    wid = lax.axis_index("s") * NC + lax.axis_index("c")
    pltpu.sync_copy(src_hbm.at[wid], src_v)
    pltpu.sync_copy(dst_hbm.at[wid], dst_v)
    pltpu.sync_copy(rln_hbm, rln_v)
    pltpu.sync_copy(rrn_hbm, rrn_v)
    pltpu.sync_copy(wsq_hbm, wsq_v)

    iota16 = lax.iota(jnp.int32, 16)
    one16 = jnp.ones((16,), jnp.int32)
    zf = jnp.zeros((16,), jnp.float32)
    w1v = [wsq_v[0, pl.ds(k * 16, 16)] for k in range(D // 16)]
    w2v = [wsq_v[1, pl.ds(k * 16, 16)] for k in range(D // 16)]

    def chunk_body(c, carry):
        cl = pltpu.async_copy(left_hbm.at[src_v.at[c]], lrows, lsem)
        cr = pltpu.async_copy(right_hbm.at[dst_v.at[c]], rrows, rsem)
        cl.wait()
        cr.wait()

        def group_body(g, carry2):
            dots1 = zf
            dots2 = zf
            for j in range(16):
                e = g * 16 + j
                a1 = [zf, zf]
                a2 = [zf, zf]
                for k in range(D // 16):
                    lv = lrows[e, pl.ds(k * 16, 16)]
                    rv = rrows[e, pl.ds(k * 16, 16)]
                    t = lv * rv
                    a1[k % 2] = a1[k % 2] + t * w1v[k]
                    a2[k % 2] = a2[k % 2] + t * w2v[k]
                lane_j = iota16 == j
                dots1 = dots1 + jnp.where(lane_j, jnp.sum(a1[0] + a1[1]), zf)
                dots2 = dots2 + jnp.where(lane_j, jnp.sum(a2[0] + a2[1]), zf)
            src2 = src_v[c, pl.ds(g * 16, 16)] * 2
            dst2 = dst_v[c, pl.ds(g * 16, 16)] * 2
            rl1 = plsc.load_gather(rln_v, [src2])
            rl2 = plsc.load_gather(rln_v, [src2 + one16])
            rr1 = plsc.load_gather(rrn_v, [dst2])
            rr2 = plsc.load_gather(rrn_v, [dst2 + one16])
            sim = 0.5 * (dots1 * rl1 * rr1 + dots2 * rl2 * rr2)
            sim = jnp.where(sim < THRESH, jnp.zeros((16,), jnp.float32), sim)
            out_v[pl.ds(c * C + g * 16, 16)] = sim
            return carry2

        return lax.fori_loop(0, NG, group_body, carry)

    lax.fori_loop(0, NCH, chunk_body, 0)
    pltpu.sync_copy(out_v, out_hbm.at[pl.ds(wid * CPW, CPW)])


_sc_call = functools.partial(
    pl.kernel,
    out_type=jax.ShapeDtypeStruct((N_EDGES,), jnp.float32),
    mesh=plsc.VectorSubcoreMesh(core_axis_name="c", subcore_axis_name="s"),
    compiler_params=pltpu.CompilerParams(needs_layout_passes=False),
    scratch_types=[
        pltpu.VMEM((NCH, C), jnp.int32),
        pltpu.VMEM((NCH, C), jnp.int32),
        pltpu.VMEM((CPW,), jnp.float32),
        pltpu.VMEM((N_NODES * NUM_HEADS,), jnp.float32),
        pltpu.VMEM((N_NODES * NUM_HEADS,), jnp.float32),
        pltpu.VMEM((NUM_HEADS, D), jnp.float32),
        pltpu.VMEM((C, D), jnp.float32),
        pltpu.VMEM((C, D), jnp.float32),
        pltpu.SemaphoreType.DMA,
        pltpu.SemaphoreType.DMA,
    ],
)(_sc_body)


def kernel(left_features, right_features, edge_index, W):
    rln, rrn, wsq = _prep(left_features, right_features, W)
    src = edge_index[0].reshape(NW, NCH, C)
    dst = edge_index[1].reshape(NW, NCH, C)
    return _sc_call(left_features, right_features, src, dst,
                    rln.reshape(-1), rrn.reshape(-1), wsq)


# speedup vs baseline: 8.8434x; 1.5883x over previous
import functools

import jax
import jax.numpy as jnp
from jax import lax
from jax.experimental import pallas as pl
from jax.experimental.pallas import tpu as pltpu
from jax.experimental.pallas import tpu_sc as plsc

N_NODES = 10000
N_EDGES = 320000
D = 128
NUM_HEADS = 2
THRESH = 0.1
EPS = 1e-8

NC = 2
NS = 16
NW = NC * NS
CPW = N_EDGES // NW
C = 80
NCH = CPW // C
NG = C // 16


def _prep_body(l_ref, r_ref, w_ref, rln_ref, rrn_ref, wsq_ref):
    w = w_ref[...]
    wsq_ref[...] = w * w
    outs = []
    for ref in (l_ref, r_ref):
        x = ref[...]
        cols = []
        for h in range(NUM_HEADS):
            xh = x * w[h:h + 1, :]
            s = jnp.sum(xh * xh, axis=1, keepdims=True)
            cols.append(lax.rsqrt(jnp.maximum(s, EPS * EPS)))
        outs.append(jnp.concatenate(cols, axis=1))
    rln_ref[...] = outs[0]
    rrn_ref[...] = outs[1]


def _prep(left, right, w):
    return pl.pallas_call(
        _prep_body,
        out_shape=(
            jax.ShapeDtypeStruct((N_NODES, NUM_HEADS), jnp.float32),
            jax.ShapeDtypeStruct((N_NODES, NUM_HEADS), jnp.float32),
            jax.ShapeDtypeStruct((NUM_HEADS, D), jnp.float32),
        ),
    )(left, right, w)


def _sc_body(left_hbm, right_hbm, src_hbm, dst_hbm, rln_hbm, rrn_hbm, wsq_hbm,
             out_hbm, src_v, dst_v, out_v, rln_v, rrn_v, wsq_v,
             lrows0, rrows0, lrows1, rrows1, lsem0, rsem0, lsem1, rsem1):
    wid = lax.axis_index("s") * NC + lax.axis_index("c")
    pltpu.sync_copy(src_hbm.at[wid], src_v)
    pltpu.sync_copy(dst_hbm.at[wid], dst_v)
    pltpu.sync_copy(rln_hbm, rln_v)
    pltpu.sync_copy(rrn_hbm, rrn_v)
    pltpu.sync_copy(wsq_hbm, wsq_v)

    iota16 = lax.iota(jnp.int32, 16)
    one16 = jnp.ones((16,), jnp.int32)
    zf = jnp.zeros((16,), jnp.float32)
    w1v = [wsq_v[0, pl.ds(k * 16, 16)] for k in range(D // 16)]
    w2v = [wsq_v[1, pl.ds(k * 16, 16)] for k in range(D // 16)]

    bufs = ((lrows0, rrows0, lsem0, rsem0), (lrows1, rrows1, lsem1, rsem1))

    def issue(c, b):
        lr, rr, ls, rs = bufs[b]
        pltpu.async_copy(left_hbm.at[src_v.at[c]], lr, ls)
        pltpu.async_copy(right_hbm.at[dst_v.at[c]], rr, rs)

    def drain(b):
        lr, rr, ls, rs = bufs[b]
        pltpu.make_async_copy(left_hbm.at[src_v.at[0]], lr, ls).wait()
        pltpu.make_async_copy(right_hbm.at[dst_v.at[0]], rr, rs).wait()

    def compute(c, b):
        lrows, rrows = bufs[b][0], bufs[b][1]

        def group_body(g, carry2):
            dots1 = zf
            dots2 = zf
            for j in range(16):
                e = g * 16 + j
                a1 = [zf, zf]
                a2 = [zf, zf]
                for k in range(D // 16):
                    lv = lrows[e, pl.ds(k * 16, 16)]
                    rv = rrows[e, pl.ds(k * 16, 16)]
                    t = lv * rv
                    a1[k % 2] = a1[k % 2] + t * w1v[k]
                    a2[k % 2] = a2[k % 2] + t * w2v[k]
                lane_j = iota16 == j
                dots1 = dots1 + jnp.where(lane_j, jnp.sum(a1[0] + a1[1]), zf)
                dots2 = dots2 + jnp.where(lane_j, jnp.sum(a2[0] + a2[1]), zf)
            src2 = src_v[c, pl.ds(g * 16, 16)] * 2
            dst2 = dst_v[c, pl.ds(g * 16, 16)] * 2
            rl1 = plsc.load_gather(rln_v, [src2])
            rl2 = plsc.load_gather(rln_v, [src2 + one16])
            rr1 = plsc.load_gather(rrn_v, [dst2])
            rr2 = plsc.load_gather(rrn_v, [dst2 + one16])
            sim = 0.5 * (dots1 * rl1 * rr1 + dots2 * rl2 * rr2)
            sim = jnp.where(sim < THRESH, jnp.zeros((16,), jnp.float32), sim)
            out_v[pl.ds(c * C + g * 16, 16)] = sim
            return carry2

        lax.fori_loop(0, NG, group_body, 0)

    issue(0, 0)
    def pair_body(i, carry):
        c0 = 2 * i
        issue(c0 + 1, 1)
        drain(0)
        compute(c0, 0)
        issue(c0 + 2, 0)
        drain(1)
        compute(c0 + 1, 1)
        return carry

    lax.fori_loop(0, (NCH - 1) // 2, pair_body, 0)
    drain(0)
    compute(NCH - 1, 0)
    pltpu.sync_copy(out_v, out_hbm.at[pl.ds(wid * CPW, CPW)])


_sc_call = functools.partial(
    pl.kernel,
    out_type=jax.ShapeDtypeStruct((N_EDGES,), jnp.float32),
    mesh=plsc.VectorSubcoreMesh(core_axis_name="c", subcore_axis_name="s"),
    compiler_params=pltpu.CompilerParams(needs_layout_passes=False),
    scratch_types=[
        pltpu.VMEM((NCH, C), jnp.int32),
        pltpu.VMEM((NCH, C), jnp.int32),
        pltpu.VMEM((CPW,), jnp.float32),
        pltpu.VMEM((N_NODES * NUM_HEADS,), jnp.float32),
        pltpu.VMEM((N_NODES * NUM_HEADS,), jnp.float32),
        pltpu.VMEM((NUM_HEADS, D), jnp.float32),
        pltpu.VMEM((C, D), jnp.float32),
        pltpu.VMEM((C, D), jnp.float32),
        pltpu.VMEM((C, D), jnp.float32),
        pltpu.VMEM((C, D), jnp.float32),
        pltpu.SemaphoreType.DMA,
        pltpu.SemaphoreType.DMA,
        pltpu.SemaphoreType.DMA,
        pltpu.SemaphoreType.DMA,
    ],
)(_sc_body)


def kernel(left_features, right_features, edge_index, W):
    rln, rrn, wsq = _prep(left_features, right_features, W)
    src = edge_index[0].reshape(NW, NCH, C)
    dst = edge_index[1].reshape(NW, NCH, C)
    return _sc_call(left_features, right_features, src, dst,
                    rln.reshape(-1), rrn.reshape(-1), wsq)
